# baseline (speedup 1.0000x reference)
import jax
import jax.numpy as jnp
from jax.experimental import pallas as pl
from jax.experimental.pallas import tpu as pltpu

_NUM_USERS = 6000
_NUM_ITEMS = 4000
_NUM_NODES = _NUM_USERS + _NUM_ITEMS
_EMBED_DIM = 256
_NUM_LAYERS = 3
_NUM_EDGES = 160000
_CHUNK = 16000
_NCHUNKS = _NUM_EDGES // _CHUNK


def _lightgcn_kernel(edges_ref, x0_ref, out_ref, xcur_ref, xnext_ref, dinv_ref):
    l = pl.program_id(0)
    c = pl.program_id(1)
    nchunks = pl.num_programs(1)
    chunk = edges_ref.shape[1]

    @pl.when(jnp.logical_and(l == 0, c == 0))
    def _init():
        dinv_ref[...] = jnp.zeros_like(dinv_ref)
        xcur_ref[...] = x0_ref[...]
        out_ref[...] = x0_ref[...]

    @pl.when(l == 0)
    def _deg_pass():
        def deg_body(e, carry):
            d = edges_ref[1, e]
            cur = dinv_ref[pl.ds(d, 1), :]
            dinv_ref[pl.ds(d, 1), :] = cur + 1.0
            return carry

        jax.lax.fori_loop(0, chunk, deg_body, 0)

    @pl.when(jnp.logical_and(l == 0, c == nchunks - 1))
    def _deg_finalize():
        deg = dinv_ref[...]
        dinv_ref[...] = jnp.where(deg > 0.0, jax.lax.rsqrt(deg), 0.0)

    @pl.when(jnp.logical_and(l > 0, c == 0))
    def _layer_init():
        xnext_ref[...] = jnp.zeros_like(xnext_ref)

    @pl.when(l > 0)
    def _layer_pass():
        def edge_body(e, carry):
            s = edges_ref[0, e]
            d = edges_ref[1, e]
            dis = dinv_ref[pl.ds(s, 1), :]
            did = dinv_ref[pl.ds(d, 1), :]
            row = xcur_ref[pl.ds(s, 1), :]
            acc = xnext_ref[pl.ds(d, 1), :]
            xnext_ref[pl.ds(d, 1), :] = acc + row * (dis * did)
            return carry

        jax.lax.fori_loop(0, chunk, edge_body, 0)

    @pl.when(jnp.logical_and(l > 0, c == nchunks - 1))
    def _layer_finalize():
        xcur_ref[...] = xnext_ref[...]
        out_ref[...] = out_ref[...] + xnext_ref[...]

    @pl.when(jnp.logical_and(l == _NUM_LAYERS, c == nchunks - 1))
    def _mean():
        out_ref[...] = out_ref[...] * (1.0 / (_NUM_LAYERS + 1))


def kernel(edge_index, user_emb_weight, item_emb_weight):
    x0 = jnp.concatenate([user_emb_weight, item_emb_weight], axis=0)
    x_final = pl.pallas_call(
        _lightgcn_kernel,
        grid=(_NUM_LAYERS + 1, _NCHUNKS),
        out_shape=jax.ShapeDtypeStruct((_NUM_NODES, _EMBED_DIM), jnp.float32),
        in_specs=[
            pl.BlockSpec((2, _CHUNK), lambda l, c: (0, c), memory_space=pltpu.SMEM),
            pl.BlockSpec((_NUM_NODES, _EMBED_DIM), lambda l, c: (0, 0)),
        ],
        out_specs=pl.BlockSpec((_NUM_NODES, _EMBED_DIM), lambda l, c: (0, 0)),
        scratch_shapes=[
            pltpu.VMEM((_NUM_NODES, _EMBED_DIM), jnp.float32),
            pltpu.VMEM((_NUM_NODES, _EMBED_DIM), jnp.float32),
            pltpu.VMEM((_NUM_NODES, 1), jnp.float32),
        ],
    )(edge_index, x0)
    return (x_final[:_NUM_USERS], x_final[_NUM_USERS:])
